# baseline (speedup 1.0000x reference)
import jax
import jax.numpy as jnp
from jax.experimental import pallas as pl
from jax.experimental.pallas import tpu as pltpu

N = 100000
DIM = 128
BLK = 12800
GRID = -(-N // BLK)
NPAD = BLK * GRID


def _body(m_ref, w_ref, x_ref, o_ref):
    i = pl.program_id(0)
    x = x_ref[...]
    moff = pl.multiple_of(i * BLK, 128)
    m = m_ref[0, 0, pl.ds(moff, BLK)].reshape(BLK, 1)
    w = w_ref[...]
    o_ref[...] = x * jnp.where(m > 0.0, w, 1.0)


def kernel(x, node_mask, deletion_weight):
    m = node_mask.astype(jnp.float32)
    m = jnp.pad(m, (0, NPAD - N)).reshape(1, 1, NPAD)
    w = deletion_weight[None, :]
    return pl.pallas_call(
        _body,
        grid=(GRID,),
        in_specs=[
            pl.BlockSpec((1, 1, NPAD), lambda i: (0, 0, 0)),
            pl.BlockSpec((1, DIM), lambda i: (0, 0)),
            pl.BlockSpec((BLK, DIM), lambda i: (i, 0)),
        ],
        out_specs=pl.BlockSpec((BLK, DIM), lambda i: (i, 0)),
        out_shape=jax.ShapeDtypeStruct((N, DIM), jnp.float32),
        compiler_params=pltpu.CompilerParams(
            dimension_semantics=("parallel",),
        ),
    )(m, w, x)
